# Initial kernel scaffold; baseline (speedup 1.0000x reference)
import jax
import jax.numpy as jnp
from jax.experimental import pallas as pl


def kernel(logits_source, labels_source, logits_target, weight):
    raise NotImplementedError("write your pallas kernel here")


# speedup vs baseline: 1.2275x; 1.2275x over previous
import functools

import jax
import jax.numpy as jnp
from jax.experimental import pallas as pl
from jax.experimental.pallas import tpu as pltpu

_N = 50000
_C = 100
_NBINS = 15
_BLK = 2000
_GRID = _N // _BLK
_PAD = 50176
_ROWS = _PAD // 128
_LO_BITS = 0x3B800000
_HI_BITS = 0x3F800000
_BISECT_ITERS = 26


def _stats_body(lab_ref, w_ref, ls_ref, lt_ref, cs_ref, aw_ref, ct_ref):
    xs = ls_ref[...]
    ms = jnp.max(xs, axis=1, keepdims=True)
    ses = jnp.sum(jnp.exp(xs - ms), axis=1, keepdims=True)
    cs_ref[...] = 1.0 / ses

    col = jax.lax.broadcasted_iota(jnp.int32, (_BLK, _C), 1)
    lab = lab_ref[...]
    first_amax = jnp.min(jnp.where(xs == ms, col, _C), axis=1, keepdims=True)
    acc = (first_amax == lab).astype(jnp.float32)
    w_at = jnp.sum(jnp.where(col == lab, w_ref[...], 0.0), axis=1, keepdims=True)
    aw_ref[...] = acc * w_at

    xt = lt_ref[...]
    mt = jnp.max(xt, axis=1, keepdims=True)
    ct_ref[...] = 1.0 / jnp.sum(jnp.exp(xt - mt), axis=1, keepdims=True)


def _count_le(keys, mid):
    return jnp.sum((keys <= mid).astype(jnp.float32))


def _select_body(kr_ref, fr_ref, ct_ref, cs_ref, aw_ref, o_ref):
    ct = ct_ref[...]
    keys = jax.lax.bitcast_convert_type(ct, jnp.int32)

    vals = []
    for j in range(32):
        k_j = kr_ref[j]

        def body(_, lh, k=k_j):
            lo, hi = lh
            mid = (lo + hi) // 2
            cnt = _count_le(keys, mid)
            ge = cnt >= k.astype(jnp.float32)
            return (jnp.where(ge, lo, mid), jnp.where(ge, mid, hi))

        _, hi = jax.lax.fori_loop(
            0, _BISECT_ITERS, body,
            (jnp.int32(_LO_BITS), jnp.int32(_HI_BITS)))
        vals.append(jax.lax.bitcast_convert_type(hi, jnp.float32))

    edges = [vals[j] + fr_ref[j] * (vals[16 + j] - vals[j]) for j in range(16)]

    cs = cs_ref[...]
    aw = aw_ref[...]
    normalizer = jnp.float32((_N - 1) / _N)
    total = jnp.float32(0.0)
    for b in range(_NBINS):
        lo_b, hi_b = edges[b], edges[b + 1]
        mt = (ct > lo_b) & (ct <= hi_b)
        cnt = jnp.sum(mt.astype(jnp.float32))
        ws = jnp.sum(jnp.where((cs > lo_b) & (cs <= hi_b), aw, 0.0))
        ce = normalizer * ws / jnp.maximum(cnt - 1.0, 1e-12)
        contrib = jnp.sum(jnp.where(mt, jnp.abs(ct - ce) ** 2, 0.0))
        total = total + jnp.where(cnt > 1.0, contrib, 0.0)
    o_ref[0, 0] = total / _N


@functools.partial(jax.jit, static_argnames=())
def _run(logits_source, labels_source, logits_target, weight):
    conf_s, aw, conf_t = pl.pallas_call(
        _stats_body,
        grid=(_GRID,),
        in_specs=[
            pl.BlockSpec((_BLK, 1), lambda i: (i, 0)),
            pl.BlockSpec((1, _C), lambda i: (0, 0)),
            pl.BlockSpec((_BLK, _C), lambda i: (i, 0)),
            pl.BlockSpec((_BLK, _C), lambda i: (i, 0)),
        ],
        out_specs=[
            pl.BlockSpec((_BLK, 1), lambda i: (i, 0)),
            pl.BlockSpec((_BLK, 1), lambda i: (i, 0)),
            pl.BlockSpec((_BLK, 1), lambda i: (i, 0)),
        ],
        out_shape=[jax.ShapeDtypeStruct((_N, 1), jnp.float32)] * 3,
    )(labels_source.reshape(_N, 1), weight.reshape(1, _C),
      logits_source, logits_target)

    xs = jnp.linspace(0.0, float(_N), _NBINS + 1)
    fl = jnp.floor(xs)
    i0 = jnp.clip(fl.astype(jnp.int32), 0, _N - 1)
    frac = jnp.where(fl >= _N, 0.0, xs - fl).astype(jnp.float32)
    i1 = jnp.clip(i0 + 1, 0, _N - 1)
    kr = jnp.concatenate([i0, i1]) + 1

    def padded(x, fill):
        return jnp.pad(x.reshape(_N), (0, _PAD - _N),
                       constant_values=fill).reshape(_ROWS, 128)

    out = pl.pallas_call(
        _select_body,
        in_specs=[
            pl.BlockSpec(memory_space=pltpu.SMEM),
            pl.BlockSpec(memory_space=pltpu.SMEM),
            pl.BlockSpec(memory_space=pltpu.VMEM),
            pl.BlockSpec(memory_space=pltpu.VMEM),
            pl.BlockSpec(memory_space=pltpu.VMEM),
        ],
        out_specs=pl.BlockSpec(memory_space=pltpu.SMEM),
        out_shape=jax.ShapeDtypeStruct((1, 1), jnp.float32),
    )(kr, frac, padded(conf_t, 2.0), padded(conf_s, 2.0), padded(aw, 0.0))
    return out.reshape(1)


def kernel(logits_source, labels_source, logits_target, weight):
    return _run(logits_source, labels_source, logits_target, weight)


# speedup vs baseline: 2.1360x; 1.7401x over previous
import functools

import jax
import jax.numpy as jnp
from jax.experimental import pallas as pl
from jax.experimental.pallas import tpu as pltpu

_N = 50000
_C = 100
_NBINS = 15
_BLK = 2000
_GRID = _N // _BLK
_PAD = 50176
_ROWS = _PAD // 128
_LO_BITS = 0x3B800000
_HI_BITS = 0x3F800000
_BISECT_ITERS = 26


_ONES = None


def _rowsum(x):
    ones = jnp.full((_C, 128), 1.0, dtype=jnp.float32)
    return jax.lax.dot_general(
        x, ones, (((1,), (0,)), ((), ())),
        preferred_element_type=jnp.float32)[:, :1]


def _stats_body(lab_ref, w_ref, ls_ref, lt_ref, cs_ref, aw_ref, ct_ref):
    xs = ls_ref[...]
    ms = jnp.max(xs, axis=1, keepdims=True)
    ses = _rowsum(jnp.exp(xs - ms))
    cs_ref[...] = 1.0 / ses

    col = jax.lax.broadcasted_iota(jnp.int32, (_BLK, _C), 1)
    lab = lab_ref[...]
    hit = ((col == lab) & (xs == ms)).astype(jnp.float32)
    aw_ref[...] = _rowsum(hit * w_ref[...])

    xt = lt_ref[...]
    mt = jnp.max(xt, axis=1, keepdims=True)
    ct_ref[...] = 1.0 / _rowsum(jnp.exp(xt - mt))


def _count_le(keys, mid):
    return jnp.sum((keys <= mid).astype(jnp.float32))


def _select_body(kr_ref, fr_ref, ct_ref, cs_ref, aw_ref, o_ref):
    ct = ct_ref[...]
    keys = jax.lax.bitcast_convert_type(ct, jnp.int32)

    ks = [kr_ref[j].astype(jnp.float32) for j in range(32)]

    def body(_, lh):
        los, his = lh
        new_los, new_his = [], []
        for j in range(32):
            mid = (los[j] + his[j]) // 2
            ge = _count_le(keys, mid) >= ks[j]
            new_los.append(jnp.where(ge, los[j], mid))
            new_his.append(jnp.where(ge, mid, his[j]))
        return (tuple(new_los), tuple(new_his))

    init = (tuple(jnp.int32(_LO_BITS) for _ in range(32)),
            tuple(jnp.int32(_HI_BITS) for _ in range(32)))
    _, his = jax.lax.fori_loop(0, _BISECT_ITERS, body, init)
    vals = [jax.lax.bitcast_convert_type(h, jnp.float32) for h in his]

    edges = [vals[j] + fr_ref[j] * (vals[16 + j] - vals[j]) for j in range(16)]

    cs = cs_ref[...]
    aw = aw_ref[...]
    normalizer = jnp.float32((_N - 1) / _N)
    total = jnp.float32(0.0)
    for b in range(_NBINS):
        lo_b, hi_b = edges[b], edges[b + 1]
        mt = (ct > lo_b) & (ct <= hi_b)
        cnt = jnp.sum(mt.astype(jnp.float32))
        ws = jnp.sum(jnp.where((cs > lo_b) & (cs <= hi_b), aw, 0.0))
        ce = normalizer * ws / jnp.maximum(cnt - 1.0, 1e-12)
        contrib = jnp.sum(jnp.where(mt, jnp.abs(ct - ce) ** 2, 0.0))
        total = total + jnp.where(cnt > 1.0, contrib, 0.0)
    o_ref[0, 0] = total / _N


@functools.partial(jax.jit, static_argnames=())
def _run(logits_source, labels_source, logits_target, weight):
    conf_s, aw, conf_t = pl.pallas_call(
        _stats_body,
        grid=(_GRID,),
        in_specs=[
            pl.BlockSpec((_BLK, 1), lambda i: (i, 0)),
            pl.BlockSpec((1, _C), lambda i: (0, 0)),
            pl.BlockSpec((_BLK, _C), lambda i: (i, 0)),
            pl.BlockSpec((_BLK, _C), lambda i: (i, 0)),
        ],
        out_specs=[
            pl.BlockSpec((_BLK, 1), lambda i: (i, 0)),
            pl.BlockSpec((_BLK, 1), lambda i: (i, 0)),
            pl.BlockSpec((_BLK, 1), lambda i: (i, 0)),
        ],
        out_shape=[jax.ShapeDtypeStruct((_N, 1), jnp.float32)] * 3,
    )(labels_source.reshape(_N, 1), weight.reshape(1, _C),
      logits_source, logits_target)

    xs = jnp.linspace(0.0, float(_N), _NBINS + 1)
    fl = jnp.floor(xs)
    i0 = jnp.clip(fl.astype(jnp.int32), 0, _N - 1)
    frac = jnp.where(fl >= _N, 0.0, xs - fl).astype(jnp.float32)
    i1 = jnp.clip(i0 + 1, 0, _N - 1)
    kr = jnp.concatenate([i0, i1]) + 1

    def padded(x, fill):
        return jnp.pad(x.reshape(_N), (0, _PAD - _N),
                       constant_values=fill).reshape(_ROWS, 128)

    out = pl.pallas_call(
        _select_body,
        in_specs=[
            pl.BlockSpec(memory_space=pltpu.SMEM),
            pl.BlockSpec(memory_space=pltpu.SMEM),
            pl.BlockSpec(memory_space=pltpu.VMEM),
            pl.BlockSpec(memory_space=pltpu.VMEM),
            pl.BlockSpec(memory_space=pltpu.VMEM),
        ],
        out_specs=pl.BlockSpec(memory_space=pltpu.SMEM),
        out_shape=jax.ShapeDtypeStruct((1, 1), jnp.float32),
    )(kr, frac, padded(conf_t, 2.0), padded(conf_s, 2.0), padded(aw, 0.0))
    return out.reshape(1)


def kernel(logits_source, labels_source, logits_target, weight):
    return _run(logits_source, labels_source, logits_target, weight)
